# baseline (speedup 1.0000x reference)
import functools

import jax
import jax.numpy as jnp
from jax import lax
from jax.experimental import pallas as pl
from jax.experimental.pallas import tpu as pltpu
from jax.experimental.pallas import tpu_sc as plsc

_B, _S, _D = 4, 8192, 768
_NW = 32
_ROWS = _S // _NW
_R = 32
_PR = 16
_NSC = _ROWS // _R
_LANES = 16
_CPR = _D // _LANES


def _sc_body(x_hbm, pos_hbm, out_hbm, *refs):
    pos_bufs = refs[0:2]
    x_bufs = refs[2:6]
    psems = refs[6:8]
    isems = refs[8:12]
    osems = refs[12:16]
    wid = lax.axis_index("c") * 16 + lax.axis_index("s")
    row0 = wid * _ROWS

    def pos_slab(k):
        return pos_hbm.at[pl.ds(row0 + k * _PR, _PR)]

    def x_slab(p, b):
        return x_hbm.at[pl.ds(b * _S + row0 + p * _R, _R)]

    def o_slab(p, b):
        return out_hbm.at[pl.ds(b * _S + row0 + p * _R, _R)]

    pltpu.async_copy(pos_slab(0), pos_bufs[0], psems[0])
    pltpu.async_copy(pos_slab(1), pos_bufs[1], psems[1])
    for b in range(_B - 1):
        pltpu.async_copy(x_slab(0, b), x_bufs[b], isems[b])

    def super_body(p, carry):
        pltpu.make_async_copy(pos_slab(2 * p), pos_bufs[0], psems[0]).wait()
        pltpu.make_async_copy(pos_slab(2 * p + 1), pos_bufs[1], psems[1]).wait()

        for b in range(_B):
            xb = x_bufs[b]
            pltpu.make_async_copy(x_slab(p, b), xb, isems[b]).wait()

            b3 = (b + 3) % _B
            if b == 0:
                @pl.when(p > 0)
                def _():
                    pltpu.make_async_copy(
                        x_bufs[b3], o_slab(p, b3), osems[b3]
                    ).wait()

                pltpu.async_copy(x_slab(p, 3), x_bufs[b3], isems[b3])
            else:
                @pl.when(p + 1 < _NSC)
                def _(b3=b3, b=b):
                    pltpu.make_async_copy(
                        x_bufs[b3], o_slab(p, b3), osems[b3]
                    ).wait()
                    pltpu.async_copy(x_slab(p + 1, b - 1), x_bufs[b3], isems[b3])

            def rowbody_lo(r, carry2, xb=xb):
                for c4 in range(_CPR):
                    sl = pl.ds(c4 * _LANES, _LANES)
                    plsc.addupdate(xb.at[r, sl], pos_bufs[0][r, sl])
                return carry2

            def rowbody_hi(r, carry2, xb=xb):
                for c4 in range(_CPR):
                    sl = pl.ds(c4 * _LANES, _LANES)
                    plsc.addupdate(xb.at[_PR + r, sl], pos_bufs[1][r, sl])
                return carry2

            lax.fori_loop(0, _PR, rowbody_lo, 0)
            if b == _B - 1:
                @pl.when(p + 1 < _NSC)
                def _():
                    pltpu.async_copy(pos_slab(2 * p + 2), pos_bufs[0], psems[0])

            lax.fori_loop(0, _PR, rowbody_hi, 0)
            if b == _B - 1:
                @pl.when(p + 1 < _NSC)
                def _():
                    pltpu.async_copy(pos_slab(2 * p + 3), pos_bufs[1], psems[1])

            pltpu.async_copy(xb, o_slab(p, b), osems[b])
        return carry

    lax.fori_loop(0, _NSC, super_body, 0)

    for b in range(_B):
        pltpu.make_async_copy(x_bufs[b], o_slab(_NSC - 1, b), osems[b]).wait()


def kernel(x, position_embeddings):
    B, S, D = x.shape
    xf = x.reshape(B * S, D)
    mesh = plsc.VectorSubcoreMesh(core_axis_name="c", subcore_axis_name="s")
    f = pl.kernel(
        _sc_body,
        mesh=mesh,
        out_type=jax.ShapeDtypeStruct((B * S, D), jnp.float32),
        scratch_types=(
            [pltpu.VMEM((_PR, _D), jnp.float32) for _ in range(2)]
            + [pltpu.VMEM((_R, _D), jnp.float32) for _ in range(4)]
            + [pltpu.SemaphoreType.DMA for _ in range(10)]
        ),
    )
    out = f(xf, position_embeddings)
    return out.reshape(B, S, D)


# speedup vs baseline: 1.3988x; 1.3988x over previous
import functools

import jax
import jax.numpy as jnp
from jax import lax
from jax.experimental import pallas as pl
from jax.experimental.pallas import tpu as pltpu
from jax.experimental.pallas import tpu_sc as plsc

_B, _S, _D = 4, 8192, 768
_NW = 32
_ROWS = _S // _NW
_R = 16
_NCH = _ROWS // _R
_LANES = 16
_CPR = _D // _LANES


def _sc_body(x_hbm, pos_hbm, out_hbm, *refs):
    pos_bufs = refs[0:2]
    x_bufs = refs[2:10]
    psems = refs[10:12]
    isems = refs[12:20]
    osems = refs[20:28]
    wid = lax.axis_index("c") * 16 + lax.axis_index("s")
    row0 = wid * _ROWS

    def pos_slab(c):
        return pos_hbm.at[pl.ds(row0 + c * _R, _R)]

    def x_slab(c, b):
        return x_hbm.at[pl.ds(b * _S + row0 + c * _R, _R)]

    def o_slab(c, b):
        return out_hbm.at[pl.ds(b * _S + row0 + c * _R, _R)]

    pltpu.async_copy(pos_slab(0), pos_bufs[0], psems[0])
    for b in range(_B):
        pltpu.async_copy(x_slab(0, b), x_bufs[b], isems[b])

    def do_chunk(c, q):
        nq = 1 - q
        pltpu.make_async_copy(pos_slab(c), pos_bufs[q], psems[q]).wait()

        @pl.when(c + 1 < _NCH)
        def _():
            pltpu.async_copy(pos_slab(c + 1), pos_bufs[nq], psems[nq])

        for b in range(_B):
            xb = x_bufs[q * _B + b]
            pltpu.make_async_copy(x_slab(c, b), xb, isems[q * _B + b]).wait()

            @pl.when((c > 0) & (c + 1 < _NCH))
            def _():
                pltpu.make_async_copy(
                    x_bufs[nq * _B + b], o_slab(c, b), osems[nq * _B + b]
                ).wait()

            @pl.when(c + 1 < _NCH)
            def _():
                pltpu.async_copy(
                    x_slab(c + 1, b), x_bufs[nq * _B + b], isems[nq * _B + b]
                )

            pb = pos_bufs[q]

            def rowbody(r, carry, xb=xb, pb=pb):
                for c4 in range(_CPR):
                    sl = pl.ds(c4 * _LANES, _LANES)
                    plsc.addupdate(xb.at[r, sl], pb[r, sl])
                return carry

            lax.fori_loop(0, _R, rowbody, 0)
            pltpu.async_copy(xb, o_slab(c, b), osems[q * _B + b])

    def pair_body(p, carry):
        do_chunk(2 * p, 0)
        do_chunk(2 * p + 1, 1)
        return carry

    lax.fori_loop(0, _NCH // 2, pair_body, 0)

    for b in range(_B):
        pltpu.make_async_copy(x_bufs[b], o_slab(_NCH - 2, b), osems[b]).wait()
        pltpu.make_async_copy(
            x_bufs[_B + b], o_slab(_NCH - 1, b), osems[_B + b]
        ).wait()


def kernel(x, position_embeddings):
    B, S, D = x.shape
    xf = x.reshape(B * S, D)
    mesh = plsc.VectorSubcoreMesh(core_axis_name="c", subcore_axis_name="s")
    f = pl.kernel(
        _sc_body,
        mesh=mesh,
        out_type=jax.ShapeDtypeStruct((B * S, D), jnp.float32),
        scratch_types=(
            [pltpu.VMEM((_R, _D), jnp.float32) for _ in range(10)]
            + [pltpu.SemaphoreType.DMA for _ in range(18)]
        ),
    )
    out = f(xf, position_embeddings)
    return out.reshape(B, S, D)


# speedup vs baseline: 1.4233x; 1.0175x over previous
import jax
import jax.numpy as jnp
from jax import lax
from jax.experimental import pallas as pl
from jax.experimental.pallas import tpu as pltpu
from jax.experimental.pallas import tpu_sc as plsc

_B, _S, _D = 4, 8192, 768
_NW = 32
_ROWS = _S // _NW
_R = 16
_NCH = _ROWS // _R
_LANES = 16
_CPR = _D // _LANES


def _sc_body(x_hbm, pos_hbm, out_hbm, *refs):
    pos_bufs = refs[0:2]
    x_bufs = refs[2:10]
    psems = refs[10:12]
    isems = refs[12:20]
    osems = refs[20:28]
    wid = lax.axis_index("c") * 16 + lax.axis_index("s")
    row0 = wid * _ROWS

    def pos_slab(c):
        return pos_hbm.at[pl.ds(row0 + c * _R, _R)]

    def x_slab(c, b):
        return x_hbm.at[pl.ds(b * _S + row0 + c * _R, _R)]

    def o_slab(c, b):
        return out_hbm.at[pl.ds(b * _S + row0 + c * _R, _R)]

    pltpu.async_copy(pos_slab(0), pos_bufs[0], psems[0])
    for b in range(_B):
        pltpu.async_copy(x_slab(0, b), x_bufs[b], isems[b])

    def do_chunk(c, q):
        nq = 1 - q
        pltpu.make_async_copy(pos_slab(c), pos_bufs[q], psems[q]).wait()

        @pl.when(c + 1 < _NCH)
        def _():
            pltpu.async_copy(pos_slab(c + 1), pos_bufs[nq], psems[nq])

        for b in range(_B):
            xb = x_bufs[q * _B + b]
            pltpu.make_async_copy(x_slab(c, b), xb, isems[q * _B + b]).wait()

            @pl.when((c > 0) & (c + 1 < _NCH))
            def _():
                pltpu.make_async_copy(
                    x_bufs[nq * _B + b], o_slab(c, b), osems[nq * _B + b]
                ).wait()

            @pl.when(c + 1 < _NCH)
            def _():
                pltpu.async_copy(
                    x_slab(c + 1, b), x_bufs[nq * _B + b], isems[nq * _B + b]
                )

            pb = pos_bufs[q]

            def rowbody(r2, carry, xb=xb, pb=pb):
                r = r2 * 2
                for dr in range(2):
                    for c4 in range(_CPR):
                        sl = pl.ds(c4 * _LANES, _LANES)
                        plsc.addupdate(xb.at[r + dr, sl], pb[r + dr, sl])
                return carry

            lax.fori_loop(0, _R // 2, rowbody, 0)
            pltpu.async_copy(xb, o_slab(c, b), osems[q * _B + b])

    def pair_body(p, carry):
        do_chunk(2 * p, 0)
        do_chunk(2 * p + 1, 1)
        return carry

    lax.fori_loop(0, _NCH // 2, pair_body, 0)

    for b in range(_B):
        pltpu.make_async_copy(x_bufs[b], o_slab(_NCH - 2, b), osems[b]).wait()
        pltpu.make_async_copy(
            x_bufs[_B + b], o_slab(_NCH - 1, b), osems[_B + b]
        ).wait()


def kernel(x, position_embeddings):
    B, S, D = x.shape
    xf = x.reshape(B * S, D)
    mesh = plsc.VectorSubcoreMesh(core_axis_name="c", subcore_axis_name="s")
    f = pl.kernel(
        _sc_body,
        mesh=mesh,
        out_type=jax.ShapeDtypeStruct((B * S, D), jnp.float32),
        scratch_types=(
            [pltpu.VMEM((_R, _D), jnp.float32) for _ in range(10)]
            + [pltpu.SemaphoreType.DMA for _ in range(18)]
        ),
    )
    out = f(xf, position_embeddings)
    return out.reshape(B, S, D)
